# baseline (speedup 1.0000x reference)
import functools

import jax
import jax.numpy as jnp
from jax import lax
from jax.experimental import pallas as pl
from jax.experimental.pallas import tpu as pltpu
from jax.experimental.pallas import tpu_sc as plsc

T = 2048
D = 1024
H = D // 2
F = 512
E = 8
TOPK = 2
B = 512
NB = (TOPK * T + E * (B - 1) + B - 1) // B
S = NB * B
NW = 32
CH = (TOPK * T) // NW
SUB = 128

_MASK_HI = -65536


def _pack(a, b):
    abits = lax.bitcast_convert_type(
        a.astype(jnp.bfloat16).astype(jnp.float32), jnp.int32)
    bbits = lax.bitcast_convert_type(
        b.astype(jnp.bfloat16).astype(jnp.float32), jnp.int32)
    word = jnp.bitwise_or(abits, lax.shift_right_logical(bbits, 16))
    return lax.bitcast_convert_type(word, jnp.float32)


def _unpack(p):
    bits = lax.bitcast_convert_type(p, jnp.int32)
    a = lax.bitcast_convert_type(jnp.bitwise_and(bits, _MASK_HI), jnp.float32)
    b = lax.bitcast_convert_type(lax.shift_left(bits, 16), jnp.float32)
    return a, b


def _unpack_bf16(p):
    a, b = _unpack(p)
    return jnp.concatenate([a, b], axis=1).astype(jnp.bfloat16)


def _router_body(x_ref, rw_ref, b_ref, w01_ref, p_ref, be_ref, xp_ref,
                 nxt_ref):
    x = x_ref[...]
    xp_ref[...] = _pack(x[:, :H], x[:, H:])
    logits = lax.dot_general(rw_ref[...], x, (((1,), (1,)), ((), ())),
                             preferred_element_type=jnp.float32)
    scores = jax.nn.sigmoid(logits)
    sc = scores + b_ref[...]
    iota = lax.broadcasted_iota(jnp.int32, (E, T), 0)
    m1 = jnp.max(sc, axis=0, keepdims=True)
    i1 = jnp.min(jnp.where(sc == m1, iota, E), axis=0, keepdims=True)
    oh1 = iota == i1
    sc2 = jnp.where(oh1, -jnp.inf, sc)
    m2 = jnp.max(sc2, axis=0, keepdims=True)
    i2 = jnp.min(jnp.where(sc2 == m2, iota, E), axis=0, keepdims=True)
    oh2 = iota == i2
    w1 = jnp.sum(jnp.where(oh1, scores, 0.0), axis=0, keepdims=True)
    w2 = jnp.sum(jnp.where(oh2, scores, 0.0), axis=0, keepdims=True)
    den = w1 + w2 + 1e-20
    w01_t = jnp.concatenate([w1 / den, w2 / den], axis=0)
    w01_ref[...] = jnp.swapaxes(w01_t, 0, 1)

    ohf = jnp.concatenate([oh1, oh2], axis=1).astype(jnp.bfloat16)
    C = 512
    ii = lax.broadcasted_iota(jnp.int32, (C, C), 0)
    jj = lax.broadcasted_iota(jnp.int32, (C, C), 1)
    triU = (ii < jj).astype(jnp.bfloat16)
    carry = jnp.zeros((E, 1), jnp.float32)
    ranks = []
    for c in range(TOPK * T // C):
        blk = ohf[:, c * C:(c + 1) * C]
        r = lax.dot_general(blk, triU, (((1,), (0,)), ((), ())),
                            preferred_element_type=jnp.float32) + carry
        ranks.append(r)
        carry = r[:, C - 1:C] + blk[:, C - 1:C].astype(jnp.float32)
    ranks = jnp.concatenate(ranks, axis=1)
    counts = carry
    nbB = jnp.floor((counts + (B - 1)) / B) * B
    ie = lax.broadcasted_iota(jnp.int32, (E, E), 0)
    je = lax.broadcasted_iota(jnp.int32, (E, E), 1)
    tri8 = (je < ie).astype(jnp.float32)
    offp = lax.dot_general(tri8, nbB, (((1,), (0,)), ((), ())),
                           preferred_element_type=jnp.float32)
    ent = jnp.sum((ranks + offp) * ohf.astype(jnp.float32), axis=0,
                  keepdims=True)
    p_ref[...] = ent.astype(jnp.int32)

    bb = (lax.broadcasted_iota(jnp.int32, (E, NB), 1) * B).astype(jnp.float32)
    active = (bb >= offp) & (bb < offp + nbB)
    eplus = lax.broadcasted_iota(jnp.int32, (E, NB), 0) + 1
    be_ref[...] = jnp.sum(jnp.where(active, eplus, 0), axis=0,
                          keepdims=True) - 1

    eio = lax.broadcasted_iota(jnp.int32, (E, NB), 0)
    cand = jnp.where((offp > bb) & (nbB > 0), eio, E)
    nxt = jnp.min(cand, axis=0, keepdims=True)
    nxt_ref[...] = jnp.where(nxt == E, -1, nxt)


def _router(x, rw, bias, *, interpret=False):
    return pl.pallas_call(
        _router_body,
        out_shape=[jax.ShapeDtypeStruct((T, 2), jnp.float32),
                   jax.ShapeDtypeStruct((1, TOPK * T), jnp.int32),
                   jax.ShapeDtypeStruct((1, NB), jnp.int32),
                   jax.ShapeDtypeStruct((T, H), jnp.float32),
                   jax.ShapeDtypeStruct((1, NB), jnp.int32)],
        interpret=interpret,
    )(x, rw, bias.reshape(E, 1))


def _sc_mesh():
    return plsc.VectorSubcoreMesh(core_axis_name="c", subcore_axis_name="s")


def _dispatch(xp, p, *, interpret=False):
    @functools.partial(
        pl.kernel,
        out_type=jax.ShapeDtypeStruct((S, H), jnp.float32),
        mesh=_sc_mesh(),
        scratch_types=[pltpu.VMEM((SUB,), jnp.int32),
                       pltpu.VMEM((SUB, H), jnp.float32),
                       pltpu.SemaphoreType.DMA],
        interpret=interpret,
    )
    def k(x_hbm, p_hbm, xg_hbm, idx_v, rows_v, sem):
        wid = lax.axis_index("s") * 2 + lax.axis_index("c")
        base = wid * CH
        for c in range(CH // SUB):
            b = base + c * SUB
            pltpu.sync_copy(p_hbm.at[pl.ds(b, SUB)], idx_v)
            t0 = lax.rem(b, T)
            pltpu.sync_copy(x_hbm.at[pl.ds(t0, SUB)], rows_v)
            pltpu.async_copy(rows_v, xg_hbm.at[idx_v], sem).wait()

    return k(xp, p)


def _shared_body(xp_ref, wg_ref, wu_ref, wd_ref, o_ref,
                 wgc_ref, wuc_ref, wdc_ref):
    @pl.when(pl.program_id(0) == 0)
    def _():
        wgc_ref[...] = wg_ref[...].astype(jnp.bfloat16)
        wuc_ref[...] = wu_ref[...].astype(jnp.bfloat16)
        wdc_ref[...] = wd_ref[...].astype(jnp.bfloat16)

    xb = _unpack_bf16(xp_ref[...])
    g = lax.dot_general(xb, wgc_ref[...], (((1,), (1,)), ((), ())),
                        preferred_element_type=jnp.float32)
    u = lax.dot_general(xb, wuc_ref[...], (((1,), (1,)), ((), ())),
                        preferred_element_type=jnp.float32)
    h = (g * jax.nn.sigmoid(g) * u).astype(jnp.bfloat16)
    o = lax.dot_general(h, wdc_ref[...], (((1,), (1,)), ((), ())),
                        preferred_element_type=jnp.float32)
    o_ref[...] = _pack(o[:, :H], o[:, H:])


def _shared(xp, wg, wu, wd, *, interpret=False):
    BT = 256
    return pl.pallas_call(
        _shared_body,
        grid=(T // BT,),
        in_specs=[pl.BlockSpec((BT, H), lambda i: (i, 0)),
                  pl.BlockSpec((F, D), lambda i: (0, 0)),
                  pl.BlockSpec((F, D), lambda i: (0, 0)),
                  pl.BlockSpec((D, F), lambda i: (0, 0))],
        out_specs=pl.BlockSpec((BT, H), lambda i: (i, 0)),
        out_shape=jax.ShapeDtypeStruct((T, H), jnp.float32),
        scratch_shapes=[pltpu.VMEM((F, D), jnp.bfloat16),
                        pltpu.VMEM((F, D), jnp.bfloat16),
                        pltpu.VMEM((D, F), jnp.bfloat16)],
        interpret=interpret,
    )(xp, wg, wu, wd)


def _wcopies(wg_ref, wu_ref, wd_ref, wgb_ref, wub_ref, wdb_ref, sems, e, b):
    return [pltpu.make_async_copy(wg_ref.at[e], wgb_ref.at[b], sems.at[b]),
            pltpu.make_async_copy(wu_ref.at[e], wub_ref.at[b], sems.at[b]),
            pltpu.make_async_copy(wd_ref.at[e], wdb_ref.at[b], sems.at[b])]


def _group_body(be_ref, nxt_ref, xg_ref, wg_ref, wu_ref, wd_ref, yg_ref,
                wgb_ref, wub_ref, wdb_ref, wgc_ref, wuc_ref, wdc_ref,
                par_ref, sems):
    i = pl.program_id(0)
    e = be_ref[i]
    boundary = jnp.logical_or(i == 0, e != be_ref[jnp.maximum(i - 1, 0)])

    @pl.when(i == 0)
    def _():
        for c in _wcopies(wg_ref, wu_ref, wd_ref, wgb_ref, wub_ref, wdb_ref,
                          sems, e, 0):
            c.start()
        par_ref[0] = 1

    @pl.when(boundary & (e >= 0))
    def _():
        b = par_ref[0] ^ 1
        for c in _wcopies(wg_ref, wu_ref, wd_ref, wgb_ref, wub_ref, wdb_ref,
                          sems, e, b):
            c.wait()
        wgc_ref[...] = wgb_ref[b].astype(jnp.bfloat16)
        wuc_ref[...] = wub_ref[b].astype(jnp.bfloat16)
        wdc_ref[...] = wdb_ref[b].astype(jnp.bfloat16)
        par_ref[0] = b
        n = nxt_ref[i]

        @pl.when(n >= 0)
        def _():
            for c in _wcopies(wg_ref, wu_ref, wd_ref, wgb_ref, wub_ref,
                              wdb_ref, sems, n, b ^ 1):
                c.start()

    @pl.when(e >= 0)
    def _():
        xb = _unpack_bf16(xg_ref[...])
        g = lax.dot_general(xb, wgc_ref[...], (((1,), (1,)), ((), ())),
                            preferred_element_type=jnp.float32)
        u = lax.dot_general(xb, wuc_ref[...], (((1,), (1,)), ((), ())),
                            preferred_element_type=jnp.float32)
        h = (g * jax.nn.sigmoid(g) * u).astype(jnp.bfloat16)
        y = lax.dot_general(h, wdc_ref[...], (((1,), (1,)), ((), ())),
                            preferred_element_type=jnp.float32)
        yg_ref[...] = _pack(y[:, :H], y[:, H:])


def _grouped(be, nxt, xg, wg, wu, wd, *, interpret=False):
    grid_spec = pltpu.PrefetchScalarGridSpec(
        num_scalar_prefetch=2,
        grid=(NB,),
        in_specs=[pl.BlockSpec((B, H), lambda i, be, nxt: (i, 0)),
                  pl.BlockSpec(memory_space=pltpu.MemorySpace.HBM),
                  pl.BlockSpec(memory_space=pltpu.MemorySpace.HBM),
                  pl.BlockSpec(memory_space=pltpu.MemorySpace.HBM)],
        out_specs=pl.BlockSpec((B, H), lambda i, be, nxt: (i, 0)),
        scratch_shapes=[pltpu.VMEM((2, F, D), jnp.float32),
                        pltpu.VMEM((2, F, D), jnp.float32),
                        pltpu.VMEM((2, D, F), jnp.float32),
                        pltpu.VMEM((F, D), jnp.bfloat16),
                        pltpu.VMEM((F, D), jnp.bfloat16),
                        pltpu.VMEM((D, F), jnp.bfloat16),
                        pltpu.SMEM((1,), jnp.int32),
                        pltpu.SemaphoreType.DMA((2,))],
    )
    return pl.pallas_call(
        _group_body,
        grid_spec=grid_spec,
        out_shape=jax.ShapeDtypeStruct((S, H), jnp.float32),
        interpret=interpret,
    )(be, nxt, xg, wg, wu, wd)


def _gather_out(yg, p, *, interpret=False):
    @functools.partial(
        pl.kernel,
        out_type=jax.ShapeDtypeStruct((TOPK * T, H), jnp.float32),
        mesh=_sc_mesh(),
        scratch_types=[pltpu.VMEM((SUB,), jnp.int32),
                       pltpu.VMEM((SUB, H), jnp.float32),
                       pltpu.SemaphoreType.DMA],
        interpret=interpret,
    )
    def k(yg_hbm, p_hbm, yh_hbm, idx_v, rows_v, sem):
        wid = lax.axis_index("s") * 2 + lax.axis_index("c")
        base = wid * CH
        for c in range(CH // SUB):
            b = base + c * SUB
            pltpu.sync_copy(p_hbm.at[pl.ds(b, SUB)], idx_v)
            pltpu.async_copy(yg_hbm.at[idx_v], rows_v, sem).wait()
            pltpu.sync_copy(rows_v, yh_hbm.at[pl.ds(b, SUB)])

    return k(yg, p)


def _final_body(sh_ref, y0_ref, y1_ref, w_ref, o_ref):
    w = w_ref[...]
    w0 = w[:, 0:1]
    w1 = w[:, 1:2]
    y0a, y0b = _unpack(y0_ref[...])
    y1a, y1b = _unpack(y1_ref[...])
    sha, shb = _unpack(sh_ref[...])
    o_ref[:, :H] = sha + w0 * y0a + w1 * y1a
    o_ref[:, H:] = shb + w0 * y0b + w1 * y1b


def _final(shared, yh, w01, *, interpret=False):
    BT = 256
    nb = T // BT
    return pl.pallas_call(
        _final_body,
        grid=(nb,),
        in_specs=[pl.BlockSpec((BT, H), lambda i: (i, 0)),
                  pl.BlockSpec((BT, H), lambda i: (i, 0)),
                  pl.BlockSpec((BT, H), lambda i: (i + nb, 0)),
                  pl.BlockSpec((BT, 2), lambda i: (i, 0))],
        out_specs=pl.BlockSpec((BT, D), lambda i: (i, 0)),
        out_shape=jax.ShapeDtypeStruct((T, D), jnp.float32),
        interpret=interpret,
    )(shared, yh, yh, w01)


def kernel(hidden_states, router_weight, e_score_correction_bias, w_gate,
           w_up, w_down, ws_gate, ws_up, ws_down):
    x = hidden_states.reshape(T, D)
    w01, p2, be2, xp, nxt2 = _router(x, router_weight,
                                     e_score_correction_bias)
    p = p2.reshape(TOPK * T)
    be = be2.reshape(NB)
    nxt = nxt2.reshape(NB)
    xg = _dispatch(xp, p)
    shared = _shared(xp, ws_gate, ws_up, ws_down)
    xg, shared = jax.lax.optimization_barrier((xg, shared))
    yg = _grouped(be, nxt, xg, w_gate, w_up, w_down)
    yh = _gather_out(yg, p)
    return _final(shared, yh, w01)


# speedup vs baseline: 1.0225x; 1.0225x over previous
import functools

import jax
import jax.numpy as jnp
from jax import lax
from jax.experimental import pallas as pl
from jax.experimental.pallas import tpu as pltpu
from jax.experimental.pallas import tpu_sc as plsc

T = 2048
D = 1024
H = D // 2
F = 512
E = 8
TOPK = 2
B = 256
NB = (TOPK * T + E * (B - 1) + B - 1) // B
S = NB * B
NW = 32
CH = (TOPK * T) // NW
SUB = 128

_MASK_HI = -65536


def _pack(a, b):
    abits = lax.bitcast_convert_type(
        a.astype(jnp.bfloat16).astype(jnp.float32), jnp.int32)
    bbits = lax.bitcast_convert_type(
        b.astype(jnp.bfloat16).astype(jnp.float32), jnp.int32)
    word = jnp.bitwise_or(abits, lax.shift_right_logical(bbits, 16))
    return lax.bitcast_convert_type(word, jnp.float32)


def _unpack(p):
    bits = lax.bitcast_convert_type(p, jnp.int32)
    a = lax.bitcast_convert_type(jnp.bitwise_and(bits, _MASK_HI), jnp.float32)
    b = lax.bitcast_convert_type(lax.shift_left(bits, 16), jnp.float32)
    return a, b


def _unpack_bf16(p):
    a, b = _unpack(p)
    return jnp.concatenate([a, b], axis=1).astype(jnp.bfloat16)


def _router_body(x_ref, rw_ref, b_ref, w01_ref, p_ref, be_ref, xp_ref,
                 nxt_ref):
    x = x_ref[...]
    xp_ref[...] = _pack(x[:, :H], x[:, H:])
    logits = lax.dot_general(rw_ref[...], x, (((1,), (1,)), ((), ())),
                             preferred_element_type=jnp.float32)
    scores = jax.nn.sigmoid(logits)
    sc = scores + b_ref[...]
    iota = lax.broadcasted_iota(jnp.int32, (E, T), 0)
    m1 = jnp.max(sc, axis=0, keepdims=True)
    i1 = jnp.min(jnp.where(sc == m1, iota, E), axis=0, keepdims=True)
    oh1 = iota == i1
    sc2 = jnp.where(oh1, -jnp.inf, sc)
    m2 = jnp.max(sc2, axis=0, keepdims=True)
    i2 = jnp.min(jnp.where(sc2 == m2, iota, E), axis=0, keepdims=True)
    oh2 = iota == i2
    w1 = jnp.sum(jnp.where(oh1, scores, 0.0), axis=0, keepdims=True)
    w2 = jnp.sum(jnp.where(oh2, scores, 0.0), axis=0, keepdims=True)
    den = w1 + w2 + 1e-20
    w01_t = jnp.concatenate([w1 / den, w2 / den], axis=0)
    w01_ref[...] = jnp.swapaxes(w01_t, 0, 1)

    ohf = jnp.concatenate([oh1, oh2], axis=1).astype(jnp.bfloat16)
    C = 512
    ii = lax.broadcasted_iota(jnp.int32, (C, C), 0)
    jj = lax.broadcasted_iota(jnp.int32, (C, C), 1)
    triU = (ii < jj).astype(jnp.bfloat16)
    carry = jnp.zeros((E, 1), jnp.float32)
    ranks = []
    for c in range(TOPK * T // C):
        blk = ohf[:, c * C:(c + 1) * C]
        r = lax.dot_general(blk, triU, (((1,), (0,)), ((), ())),
                            preferred_element_type=jnp.float32) + carry
        ranks.append(r)
        carry = r[:, C - 1:C] + blk[:, C - 1:C].astype(jnp.float32)
    ranks = jnp.concatenate(ranks, axis=1)
    counts = carry
    nbB = jnp.floor((counts + (B - 1)) / B) * B
    ie = lax.broadcasted_iota(jnp.int32, (E, E), 0)
    je = lax.broadcasted_iota(jnp.int32, (E, E), 1)
    tri8 = (je < ie).astype(jnp.float32)
    offp = lax.dot_general(tri8, nbB, (((1,), (0,)), ((), ())),
                           preferred_element_type=jnp.float32)
    ent = jnp.sum((ranks + offp) * ohf.astype(jnp.float32), axis=0,
                  keepdims=True)
    p_ref[...] = ent.astype(jnp.int32)

    bb = (lax.broadcasted_iota(jnp.int32, (E, NB), 1) * B).astype(jnp.float32)
    active = (bb >= offp) & (bb < offp + nbB)
    eplus = lax.broadcasted_iota(jnp.int32, (E, NB), 0) + 1
    be_ref[...] = jnp.sum(jnp.where(active, eplus, 0), axis=0,
                          keepdims=True) - 1

    eio = lax.broadcasted_iota(jnp.int32, (E, NB), 0)
    cand = jnp.where((offp > bb) & (nbB > 0), eio, E)
    nxt = jnp.min(cand, axis=0, keepdims=True)
    nxt_ref[...] = jnp.where(nxt == E, -1, nxt)


def _router(x, rw, bias, *, interpret=False):
    return pl.pallas_call(
        _router_body,
        out_shape=[jax.ShapeDtypeStruct((T, 2), jnp.float32),
                   jax.ShapeDtypeStruct((1, TOPK * T), jnp.int32),
                   jax.ShapeDtypeStruct((1, NB), jnp.int32),
                   jax.ShapeDtypeStruct((T, H), jnp.float32),
                   jax.ShapeDtypeStruct((1, NB), jnp.int32)],
        interpret=interpret,
    )(x, rw, bias.reshape(E, 1))


def _sc_mesh():
    return plsc.VectorSubcoreMesh(core_axis_name="c", subcore_axis_name="s")


def _dispatch(xp, p, *, interpret=False):
    @functools.partial(
        pl.kernel,
        out_type=jax.ShapeDtypeStruct((S, H), jnp.float32),
        mesh=_sc_mesh(),
        scratch_types=[pltpu.VMEM((SUB,), jnp.int32),
                       pltpu.VMEM((SUB, H), jnp.float32),
                       pltpu.SemaphoreType.DMA],
        interpret=interpret,
    )
    def k(x_hbm, p_hbm, xg_hbm, idx_v, rows_v, sem):
        wid = lax.axis_index("s") * 2 + lax.axis_index("c")
        base = wid * CH
        for c in range(CH // SUB):
            b = base + c * SUB
            pltpu.sync_copy(p_hbm.at[pl.ds(b, SUB)], idx_v)
            t0 = lax.rem(b, T)
            pltpu.sync_copy(x_hbm.at[pl.ds(t0, SUB)], rows_v)
            pltpu.async_copy(rows_v, xg_hbm.at[idx_v], sem).wait()

    return k(xp, p)


def _shared_body(xp_ref, wg_ref, wu_ref, wd_ref, o_ref,
                 wgc_ref, wuc_ref, wdc_ref):
    @pl.when(pl.program_id(0) == 0)
    def _():
        wgc_ref[...] = wg_ref[...].astype(jnp.bfloat16)
        wuc_ref[...] = wu_ref[...].astype(jnp.bfloat16)
        wdc_ref[...] = wd_ref[...].astype(jnp.bfloat16)

    xb = _unpack_bf16(xp_ref[...])
    g = lax.dot_general(xb, wgc_ref[...], (((1,), (1,)), ((), ())),
                        preferred_element_type=jnp.float32)
    u = lax.dot_general(xb, wuc_ref[...], (((1,), (1,)), ((), ())),
                        preferred_element_type=jnp.float32)
    h = (g * jax.nn.sigmoid(g) * u).astype(jnp.bfloat16)
    o = lax.dot_general(h, wdc_ref[...], (((1,), (1,)), ((), ())),
                        preferred_element_type=jnp.float32)
    o_ref[...] = _pack(o[:, :H], o[:, H:])


def _shared(xp, wg, wu, wd, *, interpret=False):
    BT = 256
    return pl.pallas_call(
        _shared_body,
        grid=(T // BT,),
        in_specs=[pl.BlockSpec((BT, H), lambda i: (i, 0)),
                  pl.BlockSpec((F, D), lambda i: (0, 0)),
                  pl.BlockSpec((F, D), lambda i: (0, 0)),
                  pl.BlockSpec((D, F), lambda i: (0, 0))],
        out_specs=pl.BlockSpec((BT, H), lambda i: (i, 0)),
        out_shape=jax.ShapeDtypeStruct((T, H), jnp.float32),
        scratch_shapes=[pltpu.VMEM((F, D), jnp.bfloat16),
                        pltpu.VMEM((F, D), jnp.bfloat16),
                        pltpu.VMEM((D, F), jnp.bfloat16)],
        interpret=interpret,
    )(xp, wg, wu, wd)


def _wcopies(wg_ref, wu_ref, wd_ref, wgb_ref, wub_ref, wdb_ref, sems, e, b):
    return [pltpu.make_async_copy(wg_ref.at[e], wgb_ref.at[b], sems.at[b]),
            pltpu.make_async_copy(wu_ref.at[e], wub_ref.at[b], sems.at[b]),
            pltpu.make_async_copy(wd_ref.at[e], wdb_ref.at[b], sems.at[b])]


def _group_body(be_ref, nxt_ref, xg_ref, wg_ref, wu_ref, wd_ref, yg_ref,
                wgb_ref, wub_ref, wdb_ref, wgc_ref, wuc_ref, wdc_ref,
                par_ref, sems):
    i = pl.program_id(0)
    e = be_ref[i]
    boundary = jnp.logical_or(i == 0, e != be_ref[jnp.maximum(i - 1, 0)])

    @pl.when(i == 0)
    def _():
        for c in _wcopies(wg_ref, wu_ref, wd_ref, wgb_ref, wub_ref, wdb_ref,
                          sems, e, 0):
            c.start()
        par_ref[0] = 1

    @pl.when(boundary & (e >= 0))
    def _():
        b = par_ref[0] ^ 1
        for c in _wcopies(wg_ref, wu_ref, wd_ref, wgb_ref, wub_ref, wdb_ref,
                          sems, e, b):
            c.wait()
        wgc_ref[...] = wgb_ref[b].astype(jnp.bfloat16)
        wuc_ref[...] = wub_ref[b].astype(jnp.bfloat16)
        wdc_ref[...] = wdb_ref[b].astype(jnp.bfloat16)
        par_ref[0] = b
        n = nxt_ref[i]

        @pl.when(n >= 0)
        def _():
            for c in _wcopies(wg_ref, wu_ref, wd_ref, wgb_ref, wub_ref,
                              wdb_ref, sems, n, b ^ 1):
                c.start()

    @pl.when(e >= 0)
    def _():
        xb = _unpack_bf16(xg_ref[...])
        g = lax.dot_general(xb, wgc_ref[...], (((1,), (1,)), ((), ())),
                            preferred_element_type=jnp.float32)
        u = lax.dot_general(xb, wuc_ref[...], (((1,), (1,)), ((), ())),
                            preferred_element_type=jnp.float32)
        h = (g * jax.nn.sigmoid(g) * u).astype(jnp.bfloat16)
        y = lax.dot_general(h, wdc_ref[...], (((1,), (1,)), ((), ())),
                            preferred_element_type=jnp.float32)
        yg_ref[...] = _pack(y[:, :H], y[:, H:])


def _grouped(be, nxt, xg, wg, wu, wd, *, interpret=False):
    grid_spec = pltpu.PrefetchScalarGridSpec(
        num_scalar_prefetch=2,
        grid=(NB,),
        in_specs=[pl.BlockSpec((B, H), lambda i, be, nxt: (i, 0)),
                  pl.BlockSpec(memory_space=pltpu.MemorySpace.HBM),
                  pl.BlockSpec(memory_space=pltpu.MemorySpace.HBM),
                  pl.BlockSpec(memory_space=pltpu.MemorySpace.HBM)],
        out_specs=pl.BlockSpec((B, H), lambda i, be, nxt: (i, 0)),
        scratch_shapes=[pltpu.VMEM((2, F, D), jnp.float32),
                        pltpu.VMEM((2, F, D), jnp.float32),
                        pltpu.VMEM((2, D, F), jnp.float32),
                        pltpu.VMEM((F, D), jnp.bfloat16),
                        pltpu.VMEM((F, D), jnp.bfloat16),
                        pltpu.VMEM((D, F), jnp.bfloat16),
                        pltpu.SMEM((1,), jnp.int32),
                        pltpu.SemaphoreType.DMA((2,))],
    )
    return pl.pallas_call(
        _group_body,
        grid_spec=grid_spec,
        out_shape=jax.ShapeDtypeStruct((S, H), jnp.float32),
        interpret=interpret,
    )(be, nxt, xg, wg, wu, wd)


def _gather_out(yg, p, *, interpret=False):
    @functools.partial(
        pl.kernel,
        out_type=jax.ShapeDtypeStruct((TOPK * T, H), jnp.float32),
        mesh=_sc_mesh(),
        scratch_types=[pltpu.VMEM((SUB,), jnp.int32),
                       pltpu.VMEM((SUB, H), jnp.float32),
                       pltpu.SemaphoreType.DMA],
        interpret=interpret,
    )
    def k(yg_hbm, p_hbm, yh_hbm, idx_v, rows_v, sem):
        wid = lax.axis_index("s") * 2 + lax.axis_index("c")
        base = wid * CH
        for c in range(CH // SUB):
            b = base + c * SUB
            pltpu.sync_copy(p_hbm.at[pl.ds(b, SUB)], idx_v)
            pltpu.async_copy(yg_hbm.at[idx_v], rows_v, sem).wait()
            pltpu.sync_copy(rows_v, yh_hbm.at[pl.ds(b, SUB)])

    return k(yg, p)


def _final_body(sh_ref, y0_ref, y1_ref, w_ref, o_ref):
    w = w_ref[...]
    w0 = w[:, 0:1]
    w1 = w[:, 1:2]
    y0a, y0b = _unpack(y0_ref[...])
    y1a, y1b = _unpack(y1_ref[...])
    sha, shb = _unpack(sh_ref[...])
    o_ref[:, :H] = sha + w0 * y0a + w1 * y1a
    o_ref[:, H:] = shb + w0 * y0b + w1 * y1b


def _final(shared, yh, w01, *, interpret=False):
    BT = 256
    nb = T // BT
    return pl.pallas_call(
        _final_body,
        grid=(nb,),
        in_specs=[pl.BlockSpec((BT, H), lambda i: (i, 0)),
                  pl.BlockSpec((BT, H), lambda i: (i, 0)),
                  pl.BlockSpec((BT, H), lambda i: (i + nb, 0)),
                  pl.BlockSpec((BT, 2), lambda i: (i, 0))],
        out_specs=pl.BlockSpec((BT, D), lambda i: (i, 0)),
        out_shape=jax.ShapeDtypeStruct((T, D), jnp.float32),
        interpret=interpret,
    )(shared, yh, yh, w01)


def kernel(hidden_states, router_weight, e_score_correction_bias, w_gate,
           w_up, w_down, ws_gate, ws_up, ws_down):
    x = hidden_states.reshape(T, D)
    w01, p2, be2, xp, nxt2 = _router(x, router_weight,
                                     e_score_correction_bias)
    p = p2.reshape(TOPK * T)
    be = be2.reshape(NB)
    nxt = nxt2.reshape(NB)
    xg = _dispatch(xp, p)
    shared = _shared(xp, ws_gate, ws_up, ws_down)
    xg, shared = jax.lax.optimization_barrier((xg, shared))
    yg = _grouped(be, nxt, xg, w_gate, w_up, w_down)
    yh = _gather_out(yg, p)
    return _final(shared, yh, w01)


# speedup vs baseline: 1.0693x; 1.0457x over previous
import functools

import jax
import jax.numpy as jnp
from jax import lax
from jax.experimental import pallas as pl
from jax.experimental.pallas import tpu as pltpu
from jax.experimental.pallas import tpu_sc as plsc

T = 2048
D = 1024
H = D // 2
F = 512
E = 8
TOPK = 2
B = 256
NB = (TOPK * T + E * (B - 1) + B - 1) // B
S = NB * B
NW = 32
CH = (TOPK * T) // NW
SUB = 128

_MASK_HI = -65536


def _pack(a, b):
    abits = lax.bitcast_convert_type(
        a.astype(jnp.bfloat16).astype(jnp.float32), jnp.int32)
    bbits = lax.bitcast_convert_type(
        b.astype(jnp.bfloat16).astype(jnp.float32), jnp.int32)
    word = jnp.bitwise_or(abits, lax.shift_right_logical(bbits, 16))
    return lax.bitcast_convert_type(word, jnp.float32)


def _unpack(p):
    bits = lax.bitcast_convert_type(p, jnp.int32)
    a = lax.bitcast_convert_type(jnp.bitwise_and(bits, _MASK_HI), jnp.float32)
    b = lax.bitcast_convert_type(lax.shift_left(bits, 16), jnp.float32)
    return a, b


def _unpack_bf16(p):
    a, b = _unpack(p)
    return jnp.concatenate([a, b], axis=1).astype(jnp.bfloat16)


def _router_body(x_ref, rw_ref, b_ref, w01_ref, p_ref, be_ref, xp_ref,
                 nxt_ref):
    x = x_ref[...]
    xp_ref[...] = _pack(x[:, :H], x[:, H:])
    logits = lax.dot_general(rw_ref[...], x, (((1,), (1,)), ((), ())),
                             preferred_element_type=jnp.float32)
    scores = jax.nn.sigmoid(logits)
    sc = scores + b_ref[...]
    iota = lax.broadcasted_iota(jnp.int32, (E, T), 0)
    m1 = jnp.max(sc, axis=0, keepdims=True)
    i1 = jnp.min(jnp.where(sc == m1, iota, E), axis=0, keepdims=True)
    oh1 = iota == i1
    sc2 = jnp.where(oh1, -jnp.inf, sc)
    m2 = jnp.max(sc2, axis=0, keepdims=True)
    i2 = jnp.min(jnp.where(sc2 == m2, iota, E), axis=0, keepdims=True)
    oh2 = iota == i2
    w1 = jnp.sum(jnp.where(oh1, scores, 0.0), axis=0, keepdims=True)
    w2 = jnp.sum(jnp.where(oh2, scores, 0.0), axis=0, keepdims=True)
    den = w1 + w2 + 1e-20
    w01_t = jnp.concatenate([w1 / den, w2 / den], axis=0)
    w01_ref[...] = jnp.swapaxes(w01_t, 0, 1)

    ohf = jnp.concatenate([oh1, oh2], axis=1).astype(jnp.bfloat16)
    C = 512
    ii = lax.broadcasted_iota(jnp.int32, (C, C), 0)
    jj = lax.broadcasted_iota(jnp.int32, (C, C), 1)
    triU = (ii < jj).astype(jnp.bfloat16)
    carry = jnp.zeros((E, 1), jnp.float32)
    ranks = []
    for c in range(TOPK * T // C):
        blk = ohf[:, c * C:(c + 1) * C]
        r = lax.dot_general(blk, triU, (((1,), (0,)), ((), ())),
                            preferred_element_type=jnp.float32) + carry
        ranks.append(r)
        carry = r[:, C - 1:C] + blk[:, C - 1:C].astype(jnp.float32)
    ranks = jnp.concatenate(ranks, axis=1)
    counts = carry
    nbB = jnp.floor((counts + (B - 1)) / B) * B
    ie = lax.broadcasted_iota(jnp.int32, (E, E), 0)
    je = lax.broadcasted_iota(jnp.int32, (E, E), 1)
    tri8 = (je < ie).astype(jnp.float32)
    offp = lax.dot_general(tri8, nbB, (((1,), (0,)), ((), ())),
                           preferred_element_type=jnp.float32)
    ent = jnp.sum((ranks + offp) * ohf.astype(jnp.float32), axis=0,
                  keepdims=True)
    p_ref[...] = ent.astype(jnp.int32)

    bb = (lax.broadcasted_iota(jnp.int32, (E, NB), 1) * B).astype(jnp.float32)
    active = (bb >= offp) & (bb < offp + nbB)
    eplus = lax.broadcasted_iota(jnp.int32, (E, NB), 0) + 1
    be_ref[...] = jnp.sum(jnp.where(active, eplus, 0), axis=0,
                          keepdims=True) - 1

    eio = lax.broadcasted_iota(jnp.int32, (E, NB), 0)
    cand = jnp.where((offp > bb) & (nbB > 0), eio, E)
    nxt = jnp.min(cand, axis=0, keepdims=True)
    nxt_ref[...] = jnp.where(nxt == E, -1, nxt)


def _router(x, rw, bias, *, interpret=False):
    return pl.pallas_call(
        _router_body,
        out_shape=[jax.ShapeDtypeStruct((T, 2), jnp.float32),
                   jax.ShapeDtypeStruct((1, TOPK * T), jnp.int32),
                   jax.ShapeDtypeStruct((1, NB), jnp.int32),
                   jax.ShapeDtypeStruct((T, H), jnp.float32),
                   jax.ShapeDtypeStruct((1, NB), jnp.int32)],
        interpret=interpret,
    )(x, rw, bias.reshape(E, 1))


def _sc_mesh():
    return plsc.VectorSubcoreMesh(core_axis_name="c", subcore_axis_name="s")


def _dispatch(xp, p, *, interpret=False):
    @functools.partial(
        pl.kernel,
        out_type=jax.ShapeDtypeStruct((S, H), jnp.float32),
        mesh=_sc_mesh(),
        scratch_types=[pltpu.VMEM((SUB,), jnp.int32),
                       pltpu.VMEM((SUB, H), jnp.float32),
                       pltpu.SemaphoreType.DMA],
        interpret=interpret,
    )
    def k(x_hbm, p_hbm, xg_hbm, idx_v, rows_v, sem):
        wid = lax.axis_index("s") * 2 + lax.axis_index("c")
        base = wid * CH
        for c in range(CH // SUB):
            b = base + c * SUB
            pltpu.sync_copy(p_hbm.at[pl.ds(b, SUB)], idx_v)
            t0 = lax.rem(b, T)
            pltpu.sync_copy(x_hbm.at[pl.ds(t0, SUB)], rows_v)
            pltpu.async_copy(rows_v, xg_hbm.at[idx_v], sem).wait()

    return k(xp, p)


def _shared_body(xp_ref, wg_ref, wu_ref, wd_ref, o_ref,
                 wgc_ref, wuc_ref, wdc_ref):
    @pl.when(pl.program_id(0) == 0)
    def _():
        wgc_ref[...] = wg_ref[...].astype(jnp.bfloat16)
        wuc_ref[...] = wu_ref[...].astype(jnp.bfloat16)
        wdc_ref[...] = wd_ref[...].astype(jnp.bfloat16)

    xb = _unpack_bf16(xp_ref[...])
    g = lax.dot_general(xb, wgc_ref[...], (((1,), (1,)), ((), ())),
                        preferred_element_type=jnp.float32)
    u = lax.dot_general(xb, wuc_ref[...], (((1,), (1,)), ((), ())),
                        preferred_element_type=jnp.float32)
    h = (g * jax.nn.sigmoid(g) * u).astype(jnp.bfloat16)
    o = lax.dot_general(h, wdc_ref[...], (((1,), (1,)), ((), ())),
                        preferred_element_type=jnp.float32)
    o_ref[...] = _pack(o[:, :H], o[:, H:])


def _shared(xp, wg, wu, wd, *, interpret=False):
    BT = 512
    return pl.pallas_call(
        _shared_body,
        grid=(T // BT,),
        in_specs=[pl.BlockSpec((BT, H), lambda i: (i, 0)),
                  pl.BlockSpec((F, D), lambda i: (0, 0)),
                  pl.BlockSpec((F, D), lambda i: (0, 0)),
                  pl.BlockSpec((D, F), lambda i: (0, 0))],
        out_specs=pl.BlockSpec((BT, H), lambda i: (i, 0)),
        out_shape=jax.ShapeDtypeStruct((T, H), jnp.float32),
        scratch_shapes=[pltpu.VMEM((F, D), jnp.bfloat16),
                        pltpu.VMEM((F, D), jnp.bfloat16),
                        pltpu.VMEM((D, F), jnp.bfloat16)],
        interpret=interpret,
    )(xp, wg, wu, wd)


def _wcopies(wg_ref, wu_ref, wd_ref, wgb_ref, wub_ref, wdb_ref, sems, e, b):
    return [pltpu.make_async_copy(wg_ref.at[e], wgb_ref.at[b], sems.at[b]),
            pltpu.make_async_copy(wu_ref.at[e], wub_ref.at[b], sems.at[b]),
            pltpu.make_async_copy(wd_ref.at[e], wdb_ref.at[b], sems.at[b])]


def _group_body(be_ref, nxt_ref, xg_ref, wg_ref, wu_ref, wd_ref, yg_ref,
                wgb_ref, wub_ref, wdb_ref, wgc_ref, wuc_ref, wdc_ref,
                par_ref, sems):
    i = pl.program_id(0)
    e = be_ref[i]
    boundary = jnp.logical_or(i == 0, e != be_ref[jnp.maximum(i - 1, 0)])

    @pl.when(i == 0)
    def _():
        for c in _wcopies(wg_ref, wu_ref, wd_ref, wgb_ref, wub_ref, wdb_ref,
                          sems, e, 0):
            c.start()
        par_ref[0] = 1

    @pl.when(boundary & (e >= 0))
    def _():
        b = par_ref[0] ^ 1
        for c in _wcopies(wg_ref, wu_ref, wd_ref, wgb_ref, wub_ref, wdb_ref,
                          sems, e, b):
            c.wait()
        wgc_ref[...] = wgb_ref[b].astype(jnp.bfloat16)
        wuc_ref[...] = wub_ref[b].astype(jnp.bfloat16)
        wdc_ref[...] = wdb_ref[b].astype(jnp.bfloat16)
        par_ref[0] = b
        n = nxt_ref[i]

        @pl.when(n >= 0)
        def _():
            for c in _wcopies(wg_ref, wu_ref, wd_ref, wgb_ref, wub_ref,
                              wdb_ref, sems, n, b ^ 1):
                c.start()

    @pl.when(e >= 0)
    def _():
        xb = _unpack_bf16(xg_ref[...])
        g = lax.dot_general(xb, wgc_ref[...], (((1,), (1,)), ((), ())),
                            preferred_element_type=jnp.float32)
        u = lax.dot_general(xb, wuc_ref[...], (((1,), (1,)), ((), ())),
                            preferred_element_type=jnp.float32)
        h = (g * jax.nn.sigmoid(g) * u).astype(jnp.bfloat16)
        y = lax.dot_general(h, wdc_ref[...], (((1,), (1,)), ((), ())),
                            preferred_element_type=jnp.float32)
        yg_ref[...] = _pack(y[:, :H], y[:, H:])


def _grouped(be, nxt, xg, wg, wu, wd, *, interpret=False):
    grid_spec = pltpu.PrefetchScalarGridSpec(
        num_scalar_prefetch=2,
        grid=(NB,),
        in_specs=[pl.BlockSpec((B, H), lambda i, be, nxt: (i, 0)),
                  pl.BlockSpec(memory_space=pltpu.MemorySpace.HBM),
                  pl.BlockSpec(memory_space=pltpu.MemorySpace.HBM),
                  pl.BlockSpec(memory_space=pltpu.MemorySpace.HBM)],
        out_specs=pl.BlockSpec((B, H), lambda i, be, nxt: (i, 0)),
        scratch_shapes=[pltpu.VMEM((2, F, D), jnp.float32),
                        pltpu.VMEM((2, F, D), jnp.float32),
                        pltpu.VMEM((2, D, F), jnp.float32),
                        pltpu.VMEM((F, D), jnp.bfloat16),
                        pltpu.VMEM((F, D), jnp.bfloat16),
                        pltpu.VMEM((D, F), jnp.bfloat16),
                        pltpu.SMEM((1,), jnp.int32),
                        pltpu.SemaphoreType.DMA((2,))],
    )
    return pl.pallas_call(
        _group_body,
        grid_spec=grid_spec,
        out_shape=jax.ShapeDtypeStruct((S, H), jnp.float32),
        interpret=interpret,
    )(be, nxt, xg, wg, wu, wd)


def _gather_out(yg, p, *, interpret=False):
    @functools.partial(
        pl.kernel,
        out_type=jax.ShapeDtypeStruct((TOPK * T, H), jnp.float32),
        mesh=_sc_mesh(),
        scratch_types=[pltpu.VMEM((SUB,), jnp.int32),
                       pltpu.VMEM((SUB, H), jnp.float32),
                       pltpu.SemaphoreType.DMA],
        interpret=interpret,
    )
    def k(yg_hbm, p_hbm, yh_hbm, idx_v, rows_v, sem):
        wid = lax.axis_index("s") * 2 + lax.axis_index("c")
        base = wid * CH
        for c in range(CH // SUB):
            b = base + c * SUB
            pltpu.sync_copy(p_hbm.at[pl.ds(b, SUB)], idx_v)
            pltpu.async_copy(yg_hbm.at[idx_v], rows_v, sem).wait()
            pltpu.sync_copy(rows_v, yh_hbm.at[pl.ds(b, SUB)])

    return k(yg, p)


def _final_body(sh_ref, y0_ref, y1_ref, w_ref, o_ref):
    w = w_ref[...]
    w0 = w[:, 0:1]
    w1 = w[:, 1:2]
    y0a, y0b = _unpack(y0_ref[...])
    y1a, y1b = _unpack(y1_ref[...])
    sha, shb = _unpack(sh_ref[...])
    o_ref[:, :H] = sha + w0 * y0a + w1 * y1a
    o_ref[:, H:] = shb + w0 * y0b + w1 * y1b


def _final(shared, yh, w01, *, interpret=False):
    BT = 512
    nb = T // BT
    return pl.pallas_call(
        _final_body,
        grid=(nb,),
        in_specs=[pl.BlockSpec((BT, H), lambda i: (i, 0)),
                  pl.BlockSpec((BT, H), lambda i: (i, 0)),
                  pl.BlockSpec((BT, H), lambda i: (i + nb, 0)),
                  pl.BlockSpec((BT, 2), lambda i: (i, 0))],
        out_specs=pl.BlockSpec((BT, D), lambda i: (i, 0)),
        out_shape=jax.ShapeDtypeStruct((T, D), jnp.float32),
        interpret=interpret,
    )(shared, yh, yh, w01)


def kernel(hidden_states, router_weight, e_score_correction_bias, w_gate,
           w_up, w_down, ws_gate, ws_up, ws_down):
    x = hidden_states.reshape(T, D)
    w01, p2, be2, xp, nxt2 = _router(x, router_weight,
                                     e_score_correction_bias)
    p = p2.reshape(TOPK * T)
    be = be2.reshape(NB)
    nxt = nxt2.reshape(NB)
    xg = _dispatch(xp, p)
    shared = _shared(xp, ws_gate, ws_up, ws_down)
    xg, shared = jax.lax.optimization_barrier((xg, shared))
    yg = _grouped(be, nxt, xg, w_gate, w_up, w_down)
    yh = _gather_out(yg, p)
    return _final(shared, yh, w01)
